# baseline (speedup 1.0000x reference)
import functools

import jax
import jax.numpy as jnp
from jax import lax
from jax.experimental import pallas as pl
from jax.experimental.pallas import tpu as pltpu
from jax.experimental.pallas import tpu_sc as plsc

K = 16
IDXW = 24
NW = 32


def _knn_body(n_valid, rows_per_blk, npad, rows_ref, coords_ref, out_ref,
              d2_ref):
    r = pl.program_id(1)
    xs = coords_ref[0, 0, :][None, :]
    ys = coords_ref[0, 1, :][None, :]
    zs = coords_ref[0, 2, :][None, :]
    rows = rows_ref[0, 0]

    sq_c = xs * xs + ys * ys + zs * zs
    sq_r = jnp.sum(rows * rows, axis=1)[:, None]
    dot = jax.lax.dot_general(
        rows, coords_ref[0], (((1,), (0,)), ((), ())),
        preferred_element_type=jnp.float32)
    d2 = sq_r + sq_c - 2.0 * dot

    col = jax.lax.broadcasted_iota(jnp.int32, (rows_per_blk, npad), 1)
    row_g = r * rows_per_blk + jax.lax.broadcasted_iota(
        jnp.int32, (rows_per_blk, npad), 0)
    inf = jnp.float32(jnp.inf)
    d2 = jnp.where((col == row_g) | (col >= n_valid), inf, d2)
    d2_ref[...] = d2

    big = jnp.int32(npad + 1)
    mp = jnp.full((rows_per_blk, 1), -jnp.inf, jnp.float32)
    ip = jnp.full((rows_per_blk, 1), -1, jnp.int32)
    for t in range(K):
        d2 = d2_ref[...]
        live = (d2 > mp) | ((d2 == mp) & (col > ip))
        dlive = jnp.where(live, d2, inf)
        m = jnp.min(dlive, axis=1)[:, None]
        idx = jnp.min(jnp.where(dlive == m, col, big), axis=1)
        out_ref[0, :, t] = idx
        mp = m
        ip = idx[:, None]


def _knn_indices(coords_batch, rows_per_blk=256):
    b, n, _ = coords_batch.shape
    npad = ((n + rows_per_blk - 1) // rows_per_blk) * rows_per_blk
    coords_t = jnp.transpose(coords_batch, (0, 2, 1))
    coords_t = jnp.pad(coords_t, ((0, 0), (0, 0), (0, npad - n)))
    nblk = npad // rows_per_blk
    coords_pad = jnp.pad(coords_batch, ((0, 0), (0, npad - n), (0, 0)))
    rows_in = coords_pad.reshape(b, nblk, rows_per_blk, 3)
    out = pl.pallas_call(
        functools.partial(_knn_body, n, rows_per_blk, npad),
        grid=(b, nblk),
        in_specs=[
            pl.BlockSpec((1, 1, rows_per_blk, 3),
                         lambda bi, ri: (bi, ri, 0, 0)),
            pl.BlockSpec((1, 3, npad), lambda bi, ri: (bi, 0, 0)),
        ],
        out_specs=pl.BlockSpec((1, rows_per_blk, K), lambda bi, ri: (bi, ri, 0)),
        out_shape=jax.ShapeDtypeStruct((b, npad, K), jnp.int32),
        scratch_shapes=[pltpu.VMEM((rows_per_blk, npad), jnp.float32)],
    )(rows_in, coords_t)
    return out[:, :n, :]


def _mm1_body(x_ref, w_ref, o_ref):
    o_ref[...] = jax.lax.dot_general(
        x_ref[...], w_ref[...], (((1,), (0,)), ((), ())),
        preferred_element_type=jnp.float32)


def _mm1(x, w, blk=2048):
    m = x.shape[0]
    f = w.shape[1]
    return pl.pallas_call(
        _mm1_body,
        grid=(m // blk,),
        in_specs=[
            pl.BlockSpec((blk, x.shape[1]), lambda i: (i, 0)),
            pl.BlockSpec(w.shape, lambda i: (0, 0)),
        ],
        out_specs=pl.BlockSpec((blk, f), lambda i: (i, 0)),
        out_shape=jax.ShapeDtypeStruct((m, f), jnp.float32),
    )(x, w)


def _mlp_body(t_ref, w2_ref, b2_ref, wf_ref, bf_ref, o_ref):
    h = jax.lax.dot_general(
        t_ref[...], w2_ref[...], (((1,), (0,)), ((), ())),
        preferred_element_type=jnp.float32)
    h = jnp.maximum(h + b2_ref[...][None, :], 0.0)
    o = jax.lax.dot_general(
        h, wf_ref[...], (((1,), (0,)), ((), ())),
        preferred_element_type=jnp.float32)
    o_ref[...] = o + bf_ref[...][None, :]


def _mlp(t, w2s, b2, wf, bf, blk=2048):
    m = t.shape[0]
    f = wf.shape[1]
    return pl.pallas_call(
        _mlp_body,
        grid=(m // blk,),
        in_specs=[
            pl.BlockSpec((blk, t.shape[1]), lambda i: (i, 0)),
            pl.BlockSpec(w2s.shape, lambda i: (0, 0)),
            pl.BlockSpec(b2.shape, lambda i: (0,)),
            pl.BlockSpec(wf.shape, lambda i: (0, 0)),
            pl.BlockSpec(bf.shape, lambda i: (0,)),
        ],
        out_specs=pl.BlockSpec((blk, f), lambda i: (i, 0)),
        out_shape=jax.ShapeDtypeStruct((m, f), jnp.float32),
    )(t, w2s, b2, wf, bf)


def _make_sc_agg(mp, d, relu_bias, gn=8):
    per = mp // NW
    nv = d // 16
    nidx = gn * K
    nch = per // gn
    inv = float(1.0 / (K + 1))
    mesh = plsc.VectorSubcoreMesh(core_axis_name="c", subcore_axis_name="s")

    def body(*refs):
        if relu_bias:
            (x_hbm, idx_hbm, b_hbm, out_hbm,
             idx_v, xloc_v, out_v, rows_a, rows_b, bias_v,
             sem_a, sem_b, sem_x) = refs
        else:
            (x_hbm, idx_hbm, out_hbm,
             idx_v, xloc_v, out_v, rows_a, rows_b,
             sem_a, sem_b, sem_x) = refs
        wid = lax.axis_index("s") * 2 + lax.axis_index("c")
        base = wid * per
        pltpu.sync_copy(idx_hbm.at[pl.ds(base * K, per * K)], idx_v)
        xcp = pltpu.async_copy(x_hbm.at[pl.ds(base, per)], xloc_v, sem_x)
        if relu_bias:
            pltpu.sync_copy(b_hbm, bias_v)
        pltpu.async_copy(x_hbm.at[idx_v.at[pl.ds(0, nidx)]], rows_a, sem_a)
        xcp.wait()

        def process(g, buf):
            for i in range(gn):
                n = g * gn + i
                for t in range(nv):
                    sl = pl.ds(16 * t, 16)
                    acc = xloc_v[n, sl]
                    for j in range(K):
                        acc = acc + buf[i * K + j, sl]
                    if relu_bias:
                        acc = jnp.maximum(acc * inv + bias_v[sl], 0.0)
                    out_v[n, sl] = acc

        def pair(p, carry):
            g0 = 2 * p
            pltpu.async_copy(
                x_hbm.at[idx_v.at[pl.ds((g0 + 1) * nidx, nidx)]],
                rows_b, sem_b)
            pltpu.make_async_copy(x_hbm.at[pl.ds(0, nidx)], rows_a,
                                  sem_a).wait()
            process(g0, rows_a)

            @pl.when(p < nch // 2 - 1)
            def _():
                pltpu.async_copy(
                    x_hbm.at[idx_v.at[pl.ds((g0 + 2) * nidx, nidx)]],
                    rows_a, sem_a)

            pltpu.make_async_copy(x_hbm.at[pl.ds(0, nidx)], rows_b,
                                  sem_b).wait()
            process(g0 + 1, rows_b)
            return carry

        lax.fori_loop(0, nch // 2, pair, 0)
        pltpu.sync_copy(out_v, out_hbm.at[pl.ds(base, per)])

    scratch = [
        pltpu.VMEM((per * K,), jnp.int32),
        pltpu.VMEM((per, d), jnp.float32),
        pltpu.VMEM((per, d), jnp.float32),
        pltpu.VMEM((nidx, d), jnp.float32),
        pltpu.VMEM((nidx, d), jnp.float32),
    ]
    if relu_bias:
        scratch.append(pltpu.VMEM((d,), jnp.float32))
    scratch += [pltpu.SemaphoreType.DMA] * 3
    return pl.kernel(
        body,
        out_type=jax.ShapeDtypeStruct((mp, d), jnp.float32),
        mesh=mesh,
        scratch_types=scratch,
        compiler_params=pltpu.CompilerParams(use_tc_tiling_on_sc=False),
    )


def kernel(coords_batch, W1, b1, W2, b2, Wf, bf):
    b, n, _ = coords_batch.shape
    nbr = _knn_indices(coords_batch)
    nbr_g = nbr + (jnp.arange(b, dtype=jnp.int32) * n)[:, None, None]
    nbr_g = nbr_g.reshape(b * n, K)

    m = b * n
    mpad = ((m + 2047) // 2048) * 2048
    coords_flat = coords_batch.reshape(m, 3)
    coords_flat = jnp.pad(coords_flat, ((0, mpad - m), (0, 0)))
    idx = jnp.pad(nbr_g, ((0, mpad - m), (0, 0))).reshape(mpad * K)

    y1 = _mm1(coords_flat, W1)
    inv = jnp.float32(1.0 / (K + 1))
    f = W1.shape[1]
    h1 = _make_sc_agg(mpad, f, True)(y1, idx, b1)
    t = _make_sc_agg(mpad, f, False)(h1, idx)
    out = _mlp(t, W2 * inv, b2, Wf, bf)
    return out[:m].reshape(b, n, Wf.shape[1])


# speedup vs baseline: 1.6155x; 1.6155x over previous
import functools

import jax
import jax.numpy as jnp
from jax import lax
from jax.experimental import pallas as pl
from jax.experimental.pallas import tpu as pltpu
from jax.experimental.pallas import tpu_sc as plsc

K = 16
IDXW = 24
NW = 32


def _knn_body(n_valid, rows_per_blk, npad, rows_ref, coords_ref, out_ref,
              d2_ref):
    r = pl.program_id(1)
    xs = coords_ref[0, 0, :][None, :]
    ys = coords_ref[0, 1, :][None, :]
    zs = coords_ref[0, 2, :][None, :]
    rows = rows_ref[0, 0]

    sq_c = xs * xs + ys * ys + zs * zs
    sq_r = jnp.sum(rows * rows, axis=1)[:, None]
    dot = jax.lax.dot_general(
        rows, coords_ref[0], (((1,), (0,)), ((), ())),
        preferred_element_type=jnp.float32)
    d2 = sq_r + sq_c - 2.0 * dot

    col = jax.lax.broadcasted_iota(jnp.int32, (rows_per_blk, npad), 1)
    row_g = r * rows_per_blk + jax.lax.broadcasted_iota(
        jnp.int32, (rows_per_blk, npad), 0)
    inf = jnp.float32(jnp.inf)
    d2 = jnp.where((col == row_g) | (col >= n_valid), inf, d2)
    d2_ref[...] = d2

    colf = col.astype(jnp.float32)
    big = jnp.float32(npad + 1)
    for t in range(K):
        d2 = d2_ref[...]
        m = jnp.min(d2, axis=1)[:, None]
        cand = jnp.where(d2 == m, colf, big)
        idxf = jnp.min(cand, axis=1)[:, None]
        out_ref[0, :, t] = idxf[:, 0].astype(jnp.int32)
        d2_ref[...] = jnp.where(colf == idxf, inf, d2)


def _knn_indices(coords_batch, rows_per_blk=256):
    b, n, _ = coords_batch.shape
    npad = ((n + rows_per_blk - 1) // rows_per_blk) * rows_per_blk
    coords_t = jnp.transpose(coords_batch, (0, 2, 1))
    coords_t = jnp.pad(coords_t, ((0, 0), (0, 0), (0, npad - n)))
    nblk = npad // rows_per_blk
    coords_pad = jnp.pad(coords_batch, ((0, 0), (0, npad - n), (0, 0)))
    rows_in = coords_pad.reshape(b, nblk, rows_per_blk, 3)
    out = pl.pallas_call(
        functools.partial(_knn_body, n, rows_per_blk, npad),
        grid=(b, nblk),
        in_specs=[
            pl.BlockSpec((1, 1, rows_per_blk, 3),
                         lambda bi, ri: (bi, ri, 0, 0)),
            pl.BlockSpec((1, 3, npad), lambda bi, ri: (bi, 0, 0)),
        ],
        out_specs=pl.BlockSpec((1, rows_per_blk, K), lambda bi, ri: (bi, ri, 0)),
        out_shape=jax.ShapeDtypeStruct((b, npad, K), jnp.int32),
        scratch_shapes=[pltpu.VMEM((rows_per_blk, npad), jnp.float32)],
    )(rows_in, coords_t)
    return out[:, :n, :]


def _mm1_body(x_ref, w_ref, o_ref):
    o_ref[...] = jax.lax.dot_general(
        x_ref[...], w_ref[...], (((1,), (0,)), ((), ())),
        preferred_element_type=jnp.float32)


def _mm1(x, w, blk=2048):
    m = x.shape[0]
    f = w.shape[1]
    return pl.pallas_call(
        _mm1_body,
        grid=(m // blk,),
        in_specs=[
            pl.BlockSpec((blk, x.shape[1]), lambda i: (i, 0)),
            pl.BlockSpec(w.shape, lambda i: (0, 0)),
        ],
        out_specs=pl.BlockSpec((blk, f), lambda i: (i, 0)),
        out_shape=jax.ShapeDtypeStruct((m, f), jnp.float32),
    )(x, w)


def _mlp_body(t_ref, w2_ref, b2_ref, wf_ref, bf_ref, o_ref):
    h = jax.lax.dot_general(
        t_ref[...], w2_ref[...], (((1,), (0,)), ((), ())),
        preferred_element_type=jnp.float32)
    h = jnp.maximum(h + b2_ref[...][None, :], 0.0)
    o = jax.lax.dot_general(
        h, wf_ref[...], (((1,), (0,)), ((), ())),
        preferred_element_type=jnp.float32)
    o_ref[...] = o + bf_ref[...][None, :]


def _mlp(t, w2s, b2, wf, bf, blk=2048):
    m = t.shape[0]
    f = wf.shape[1]
    return pl.pallas_call(
        _mlp_body,
        grid=(m // blk,),
        in_specs=[
            pl.BlockSpec((blk, t.shape[1]), lambda i: (i, 0)),
            pl.BlockSpec(w2s.shape, lambda i: (0, 0)),
            pl.BlockSpec(b2.shape, lambda i: (0,)),
            pl.BlockSpec(wf.shape, lambda i: (0, 0)),
            pl.BlockSpec(bf.shape, lambda i: (0,)),
        ],
        out_specs=pl.BlockSpec((blk, f), lambda i: (i, 0)),
        out_shape=jax.ShapeDtypeStruct((m, f), jnp.float32),
    )(t, w2s, b2, wf, bf)


def _make_sc_agg(mp, d, relu_bias, gn=8):
    per = mp // NW
    nv = d // 16
    nidx = gn * K
    nch = per // gn
    inv = float(1.0 / (K + 1))
    mesh = plsc.VectorSubcoreMesh(core_axis_name="c", subcore_axis_name="s")

    def body(*refs):
        if relu_bias:
            (x_hbm, idx_hbm, b_hbm, out_hbm,
             idx_v, xloc_v, out_v, rows_a, rows_b, bias_v,
             sem_a, sem_b, sem_x) = refs
        else:
            (x_hbm, idx_hbm, out_hbm,
             idx_v, xloc_v, out_v, rows_a, rows_b,
             sem_a, sem_b, sem_x) = refs
        wid = lax.axis_index("s") * 2 + lax.axis_index("c")
        base = wid * per
        pltpu.sync_copy(idx_hbm.at[pl.ds(base * K, per * K)], idx_v)
        xcp = pltpu.async_copy(x_hbm.at[pl.ds(base, per)], xloc_v, sem_x)
        if relu_bias:
            pltpu.sync_copy(b_hbm, bias_v)
        pltpu.async_copy(x_hbm.at[idx_v.at[pl.ds(0, nidx)]], rows_a, sem_a)
        xcp.wait()

        def process(g, buf):
            for i in range(gn):
                n = g * gn + i
                for t in range(nv):
                    sl = pl.ds(16 * t, 16)
                    acc = xloc_v[n, sl]
                    for j in range(K):
                        acc = acc + buf[i * K + j, sl]
                    if relu_bias:
                        acc = jnp.maximum(acc * inv + bias_v[sl], 0.0)
                    out_v[n, sl] = acc

        def pair(p, carry):
            g0 = 2 * p
            pltpu.async_copy(
                x_hbm.at[idx_v.at[pl.ds((g0 + 1) * nidx, nidx)]],
                rows_b, sem_b)
            pltpu.make_async_copy(x_hbm.at[pl.ds(0, nidx)], rows_a,
                                  sem_a).wait()
            process(g0, rows_a)

            @pl.when(p < nch // 2 - 1)
            def _():
                pltpu.async_copy(
                    x_hbm.at[idx_v.at[pl.ds((g0 + 2) * nidx, nidx)]],
                    rows_a, sem_a)

            pltpu.make_async_copy(x_hbm.at[pl.ds(0, nidx)], rows_b,
                                  sem_b).wait()
            process(g0 + 1, rows_b)
            return carry

        lax.fori_loop(0, nch // 2, pair, 0)
        pltpu.sync_copy(out_v, out_hbm.at[pl.ds(base, per)])

    scratch = [
        pltpu.VMEM((per * K,), jnp.int32),
        pltpu.VMEM((per, d), jnp.float32),
        pltpu.VMEM((per, d), jnp.float32),
        pltpu.VMEM((nidx, d), jnp.float32),
        pltpu.VMEM((nidx, d), jnp.float32),
    ]
    if relu_bias:
        scratch.append(pltpu.VMEM((d,), jnp.float32))
    scratch += [pltpu.SemaphoreType.DMA] * 3
    return pl.kernel(
        body,
        out_type=jax.ShapeDtypeStruct((mp, d), jnp.float32),
        mesh=mesh,
        scratch_types=scratch,
        compiler_params=pltpu.CompilerParams(use_tc_tiling_on_sc=False),
    )


def kernel(coords_batch, W1, b1, W2, b2, Wf, bf):
    b, n, _ = coords_batch.shape
    nbr = _knn_indices(coords_batch)
    nbr_g = nbr + (jnp.arange(b, dtype=jnp.int32) * n)[:, None, None]
    nbr_g = nbr_g.reshape(b * n, K)

    m = b * n
    mpad = ((m + 2047) // 2048) * 2048
    coords_flat = coords_batch.reshape(m, 3)
    coords_flat = jnp.pad(coords_flat, ((0, mpad - m), (0, 0)))
    idx = jnp.pad(nbr_g, ((0, mpad - m), (0, 0))).reshape(mpad * K)

    y1 = _mm1(coords_flat, W1)
    inv = jnp.float32(1.0 / (K + 1))
    f = W1.shape[1]
    h1 = _make_sc_agg(mpad, f, True)(y1, idx, b1)
    t = _make_sc_agg(mpad, f, False)(h1, idx)
    out = _mlp(t, W2 * inv, b2, Wf, bf)
    return out[:m].reshape(b, n, Wf.shape[1])


# speedup vs baseline: 1.6472x; 1.0196x over previous
import functools

import jax
import jax.numpy as jnp
from jax import lax
from jax.experimental import pallas as pl
from jax.experimental.pallas import tpu as pltpu
from jax.experimental.pallas import tpu_sc as plsc

K = 16
IDXW = 24
NW = 32


def _knn_body(n_valid, rows_per_blk, npad, rows_ref, coords_ref, out_ref,
              d2_ref):
    r = pl.program_id(1)
    xs = coords_ref[0, 0, :][None, :]
    ys = coords_ref[0, 1, :][None, :]
    zs = coords_ref[0, 2, :][None, :]
    rows = rows_ref[0, 0]

    sq_c = xs * xs + ys * ys + zs * zs
    sq_r = jnp.sum(rows * rows, axis=1)[:, None]
    dot = jax.lax.dot_general(
        rows, coords_ref[0], (((1,), (0,)), ((), ())),
        preferred_element_type=jnp.float32)
    d2 = sq_r + sq_c - 2.0 * dot

    col = jax.lax.broadcasted_iota(jnp.int32, (rows_per_blk, npad), 1)
    row_g = r * rows_per_blk + jax.lax.broadcasted_iota(
        jnp.int32, (rows_per_blk, npad), 0)
    inf = jnp.float32(jnp.inf)
    d2 = jnp.where((col == row_g) | (col >= n_valid), inf, d2)
    d2_ref[...] = d2

    colf = col.astype(jnp.float32)
    big = jnp.float32(npad + 1)
    for t in range(K):
        d2 = d2_ref[...]
        m = jnp.min(d2, axis=1)[:, None]
        cand = jnp.where(d2 == m, colf, big)
        idxf = jnp.min(cand, axis=1)[:, None]
        out_ref[0, :, t] = idxf[:, 0].astype(jnp.int32)
        d2_ref[...] = jnp.where(colf == idxf, inf, d2)


def _knn_indices(coords_batch, rows_per_blk=256):
    b, n, _ = coords_batch.shape
    npad = ((n + rows_per_blk - 1) // rows_per_blk) * rows_per_blk
    coords_t = jnp.transpose(coords_batch, (0, 2, 1))
    coords_t = jnp.pad(coords_t, ((0, 0), (0, 0), (0, npad - n)))
    nblk = npad // rows_per_blk
    coords_pad = jnp.pad(coords_batch, ((0, 0), (0, npad - n), (0, 0)))
    rows_in = coords_pad.reshape(b, nblk, rows_per_blk, 3)
    out = pl.pallas_call(
        functools.partial(_knn_body, n, rows_per_blk, npad),
        grid=(b, nblk),
        in_specs=[
            pl.BlockSpec((1, 1, rows_per_blk, 3),
                         lambda bi, ri: (bi, ri, 0, 0)),
            pl.BlockSpec((1, 3, npad), lambda bi, ri: (bi, 0, 0)),
        ],
        out_specs=pl.BlockSpec((1, rows_per_blk, K), lambda bi, ri: (bi, ri, 0)),
        out_shape=jax.ShapeDtypeStruct((b, npad, K), jnp.int32),
        scratch_shapes=[pltpu.VMEM((rows_per_blk, npad), jnp.float32)],
    )(rows_in, coords_t)
    return out[:, :n, :]


def _mm1_body(x_ref, w_ref, o_ref):
    o_ref[...] = jax.lax.dot_general(
        x_ref[...], w_ref[...], (((1,), (0,)), ((), ())),
        preferred_element_type=jnp.float32)


def _mm1(x, w, blk=2048):
    m = x.shape[0]
    f = w.shape[1]
    return pl.pallas_call(
        _mm1_body,
        grid=(m // blk,),
        in_specs=[
            pl.BlockSpec((blk, x.shape[1]), lambda i: (i, 0)),
            pl.BlockSpec(w.shape, lambda i: (0, 0)),
        ],
        out_specs=pl.BlockSpec((blk, f), lambda i: (i, 0)),
        out_shape=jax.ShapeDtypeStruct((m, f), jnp.float32),
    )(x, w)


def _mlp_body(t_ref, w2_ref, b2_ref, wf_ref, bf_ref, o_ref):
    h = jax.lax.dot_general(
        t_ref[...], w2_ref[...], (((1,), (0,)), ((), ())),
        preferred_element_type=jnp.float32)
    h = jnp.maximum(h + b2_ref[...][None, :], 0.0)
    o = jax.lax.dot_general(
        h, wf_ref[...], (((1,), (0,)), ((), ())),
        preferred_element_type=jnp.float32)
    o_ref[...] = o + bf_ref[...][None, :]


def _mlp(t, w2s, b2, wf, bf, blk=2048):
    m = t.shape[0]
    f = wf.shape[1]
    return pl.pallas_call(
        _mlp_body,
        grid=(m // blk,),
        in_specs=[
            pl.BlockSpec((blk, t.shape[1]), lambda i: (i, 0)),
            pl.BlockSpec(w2s.shape, lambda i: (0, 0)),
            pl.BlockSpec(b2.shape, lambda i: (0,)),
            pl.BlockSpec(wf.shape, lambda i: (0, 0)),
            pl.BlockSpec(bf.shape, lambda i: (0,)),
        ],
        out_specs=pl.BlockSpec((blk, f), lambda i: (i, 0)),
        out_shape=jax.ShapeDtypeStruct((m, f), jnp.float32),
    )(t, w2s, b2, wf, bf)


def _make_sc_agg(mp, d, relu_bias, gn=8):
    per = mp // NW
    nv = d // 16
    nidx = gn * K
    nch = per // gn
    inv = float(1.0 / (K + 1))
    mesh = plsc.VectorSubcoreMesh(core_axis_name="c", subcore_axis_name="s")

    def body(*refs):
        if relu_bias:
            (x_hbm, idx_hbm, b_hbm, out_hbm,
             idx_v, xloc_v, out_v, rows_a, rows_b, bias_v,
             sem_a, sem_b, sem_x) = refs
        else:
            (x_hbm, idx_hbm, out_hbm,
             idx_v, xloc_v, out_v, rows_a, rows_b,
             sem_a, sem_b, sem_x) = refs
        wid = lax.axis_index("s") * 2 + lax.axis_index("c")
        base = wid * per
        pltpu.sync_copy(idx_hbm.at[pl.ds(base * K, per * K)], idx_v)
        xcp = pltpu.async_copy(x_hbm.at[pl.ds(base, per)], xloc_v, sem_x)
        if relu_bias:
            pltpu.sync_copy(b_hbm, bias_v)
        pltpu.async_copy(x_hbm.at[idx_v.at[pl.ds(0, nidx)]], rows_a, sem_a)
        xcp.wait()

        def process(g, buf):
            for i in range(gn):
                n = g * gn + i
                for t in range(nv):
                    sl = pl.ds(16 * t, 16)
                    acc = xloc_v[n, sl]
                    for j in range(K):
                        acc = acc + buf[i * K + j, sl]
                    if relu_bias:
                        acc = jnp.maximum(acc * inv + bias_v[sl], 0.0)
                    out_v[n, sl] = acc

        def pair(p, carry):
            g0 = 2 * p
            pltpu.async_copy(
                x_hbm.at[idx_v.at[pl.ds((g0 + 1) * nidx, nidx)]],
                rows_b, sem_b)
            pltpu.make_async_copy(x_hbm.at[pl.ds(0, nidx)], rows_a,
                                  sem_a).wait()
            process(g0, rows_a)

            @pl.when(p < nch // 2 - 1)
            def _():
                pltpu.async_copy(
                    x_hbm.at[idx_v.at[pl.ds((g0 + 2) * nidx, nidx)]],
                    rows_a, sem_a)

            pltpu.make_async_copy(x_hbm.at[pl.ds(0, nidx)], rows_b,
                                  sem_b).wait()
            process(g0 + 1, rows_b)
            return carry

        lax.fori_loop(0, nch // 2, pair, 0)
        pltpu.sync_copy(out_v, out_hbm.at[pl.ds(base, per)])

    scratch = [
        pltpu.VMEM((per * K,), jnp.int32),
        pltpu.VMEM((per, d), jnp.float32),
        pltpu.VMEM((per, d), jnp.float32),
        pltpu.VMEM((nidx, d), jnp.float32),
        pltpu.VMEM((nidx, d), jnp.float32),
    ]
    if relu_bias:
        scratch.append(pltpu.VMEM((d,), jnp.float32))
    scratch += [pltpu.SemaphoreType.DMA] * 3
    return pl.kernel(
        body,
        out_type=jax.ShapeDtypeStruct((mp, d), jnp.float32),
        mesh=mesh,
        scratch_types=scratch,
        compiler_params=pltpu.CompilerParams(use_tc_tiling_on_sc=False),
    )


def kernel(coords_batch, W1, b1, W2, b2, Wf, bf):
    b, n, _ = coords_batch.shape
    mpad = ((n + 2047) // 2048) * 2048
    inv = jnp.float32(1.0 / (K + 1))
    f = W1.shape[1]
    w2s = W2 * inv

    outs = []
    for bi in range(b):
        nbr = _knn_indices(coords_batch[bi:bi + 1])
        idx = jnp.pad(nbr[0], ((0, mpad - n), (0, 0))).reshape(mpad * K)
        coords_flat = jnp.pad(coords_batch[bi], ((0, mpad - n), (0, 0)))
        y1 = _mm1(coords_flat, W1)
        h1 = _make_sc_agg(mpad, f, True)(y1, idx, b1)
        t = _make_sc_agg(mpad, f, False)(h1, idx)
        out = _mlp(t, w2s, b2, Wf, bf)
        outs.append(out[:n])
    return jnp.stack(outs, axis=0)
